# baseline (speedup 1.0000x reference)
import functools

import jax
import jax.numpy as jnp
from jax import lax
from jax.experimental import pallas as pl
from jax.experimental.pallas import tpu as pltpu
from jax.experimental.pallas import tpu_sc as plsc

N = 10000
E = 320000
D = 128
H = 128
C = 16

NC = 2
NS = 16
NW = NC * NS
K = 128
NCH = 80
GK = 4
NG = NCH // GK
EP = NW * NCH * K
NP = 10240
RPT = NP // NS

_MESH = dict(core_axis_name="c", subcore_axis_name="s",
             num_cores=NC, num_subcores=NS)


def _deg_call(dst3d, zpad):
    @functools.partial(
        pl.kernel,
        out_type=jax.ShapeDtypeStruct((NC, 1, NP), jnp.float32),
        mesh=plsc.VectorSubcoreMesh(**_MESH),
        scratch_types=[
            pltpu.VMEM((NG, GK, K), jnp.int32),
            pltpu.VMEM((K,), jnp.float32),
            pltpu.VMEM_SHARED((NP,), jnp.float32),
        ],
    )
    def deg_kernel(dst_hbm, zero_hbm, out_hbm, dst_v, ones_v, acc):
        cid = lax.axis_index("c")
        sid = lax.axis_index("s")
        wid = cid * NS + sid
        for j in range(K // 16):
            ones_v[pl.ds(16 * j, 16)] = jnp.full((16,), 1.0, jnp.float32)
        pltpu.sync_copy(zero_hbm.at[pl.ds(sid * RPT, RPT)],
                        acc.at[pl.ds(sid * RPT, RPT)])
        pltpu.sync_copy(dst_hbm.at[wid], dst_v)
        plsc.subcore_barrier()

        def body(g, carry):
            for j in range(GK):
                pltpu.sync_copy(ones_v, acc.at[dst_v.at[g, j]], add=True)
            return carry

        lax.fori_loop(0, NG, body, 0)
        plsc.subcore_barrier()
        pltpu.sync_copy(acc.at[pl.ds(sid * RPT, RPT)],
                        out_hbm.at[cid, 0, pl.ds(sid * RPT, RPT)])

    return deg_kernel(dst3d, zpad)


def _scatter_call(z, src3d, dst3d, zeros, width):
    @functools.partial(
        pl.kernel,
        out_type=jax.ShapeDtypeStruct((NC, NP, width), jnp.float32),
        mesh=plsc.VectorSubcoreMesh(**_MESH),
        scratch_types=[
            pltpu.VMEM((NG, GK, K), jnp.int32),
            pltpu.VMEM((NG, GK, K), jnp.int32),
            pltpu.VMEM((K, width), jnp.float32),
            pltpu.VMEM_SHARED((NP, width), jnp.float32),
            pltpu.SemaphoreType.DMA,
        ],
    )
    def scat_kernel(z_hbm, src_hbm, dst_hbm, zero_hbm, out_hbm,
                    src_v, dst_v, rows, acc, gsem):
        cid = lax.axis_index("c")
        sid = lax.axis_index("s")
        wid = cid * NS + sid

        pltpu.sync_copy(zero_hbm.at[pl.ds(sid * RPT, RPT)],
                        acc.at[pl.ds(sid * RPT, RPT)])
        pltpu.sync_copy(src_hbm.at[wid], src_v)
        pltpu.sync_copy(dst_hbm.at[wid], dst_v)
        plsc.subcore_barrier()

        def body(g, carry):
            for j in range(GK):
                pltpu.async_copy(z_hbm.at[src_v.at[g, j]], rows, gsem).wait()
                pltpu.sync_copy(rows, acc.at[dst_v.at[g, j]], add=True)
            return carry

        lax.fori_loop(0, NG, body, 0)
        plsc.subcore_barrier()
        pltpu.sync_copy(acc.at[pl.ds(sid * RPT, RPT)],
                        out_hbm.at[cid, pl.ds(sid * RPT, RPT)])

    return scat_kernel(z, src3d, dst3d, zeros)


BN = 2048


def _tc1_call(degt, xp, W1):
    def body(deg_ref, x_ref, w_ref, z_ref, dinv_ref):
        deg = deg_ref[:, 0:1] + deg_ref[:, 1:2] + 1.0
        dinv = lax.rsqrt(deg)
        dinv_ref[...] = dinv
        z_ref[...] = dinv * jnp.dot(x_ref[...], w_ref[...],
                                    preferred_element_type=jnp.float32)

    return pl.pallas_call(
        body,
        grid=(NP // BN,),
        in_specs=[
            pl.BlockSpec((BN, 2), lambda i: (i, 0)),
            pl.BlockSpec((BN, D), lambda i: (i, 0)),
            pl.BlockSpec((D, H), lambda i: (0, 0)),
        ],
        out_specs=[
            pl.BlockSpec((BN, H), lambda i: (i, 0)),
            pl.BlockSpec((BN, 1), lambda i: (i, 0)),
        ],
        out_shape=[
            jax.ShapeDtypeStruct((NP, H), jnp.float32),
            jax.ShapeDtypeStruct((NP, 1), jnp.float32),
        ],
    )(degt, xp, W1)


def _tc2_call(p, z1, dinv, b1):
    def body(p_ref, z1_ref, dinv_ref, b1_ref, y_ref):
        agg = p_ref[0] + p_ref[1] + z1_ref[...]
        h = jnp.maximum(dinv_ref[...] * agg + b1_ref[...], 0.0)
        y_ref[...] = dinv_ref[...] * h

    return pl.pallas_call(
        body,
        grid=(NP // BN,),
        in_specs=[
            pl.BlockSpec((NC, BN, H), lambda i: (0, i, 0)),
            pl.BlockSpec((BN, H), lambda i: (i, 0)),
            pl.BlockSpec((BN, 1), lambda i: (i, 0)),
            pl.BlockSpec((1, H), lambda i: (0, 0)),
        ],
        out_specs=pl.BlockSpec((BN, H), lambda i: (i, 0)),
        out_shape=jax.ShapeDtypeStruct((NP, H), jnp.float32),
    )(p, z1, dinv, b1)


def _tc3_call(q, y, dinv, W2, b2):
    def body(q_ref, y_ref, dinv_ref, w_ref, b2_ref, o_ref):
        agg = q_ref[0] + q_ref[1] + y_ref[...]
        o = dinv_ref[...] * jnp.dot(agg, w_ref[...],
                                    preferred_element_type=jnp.float32)
        o = o + b2_ref[...]
        m = jnp.max(o, axis=1, keepdims=True)
        s = jnp.log(jnp.sum(jnp.exp(o - m), axis=1, keepdims=True))
        o_ref[...] = o - m - s

    return pl.pallas_call(
        body,
        grid=(NP // BN,),
        in_specs=[
            pl.BlockSpec((NC, BN, H), lambda i: (0, i, 0)),
            pl.BlockSpec((BN, H), lambda i: (i, 0)),
            pl.BlockSpec((BN, 1), lambda i: (i, 0)),
            pl.BlockSpec((H, C), lambda i: (0, 0)),
            pl.BlockSpec((1, C), lambda i: (0, 0)),
        ],
        out_specs=pl.BlockSpec((BN, C), lambda i: (i, 0)),
        out_shape=jax.ShapeDtypeStruct((NP, C), jnp.float32),
    )(q, y, dinv, W2, b2)


def kernel(x, edge_index, W1, b1, W2, b2):
    pad = jnp.full((EP - E,), NP - 1, jnp.int32)
    src3d = jnp.concatenate([edge_index[0], pad]).reshape(NW, NG, GK, K)
    dst3d = jnp.concatenate([edge_index[1], pad]).reshape(NW, NG, GK, K)
    xp = jnp.pad(x, ((0, NP - N), (0, 0)))
    zpad = jnp.zeros((NP,), jnp.float32)
    zD = jnp.zeros((NP, H), jnp.float32)

    degp = _deg_call(dst3d, zpad)
    degt = degp.reshape(NC, NP).T
    z1, dinv = _tc1_call(degt, xp, W1)
    p = _scatter_call(z1, src3d, dst3d, zD, H)
    y = _tc2_call(p, z1, dinv, b1.reshape(1, H))
    q = _scatter_call(y, src3d, dst3d, zD, H)
    return _tc3_call(q, y, dinv, W2, b2.reshape(1, C))[:N]


# speedup vs baseline: 2.7314x; 2.7314x over previous
import functools

import jax
import jax.numpy as jnp
from jax import lax
from jax.experimental import pallas as pl
from jax.experimental.pallas import tpu as pltpu
from jax.experimental.pallas import tpu_sc as plsc

N = 10000
E = 320000
D = 128
H = 128
C = 16

NC = 2
NS = 16
NW = NC * NS
K = 128
NCH = 80
GK = 4
NG = NCH // GK
EP = NW * NCH * K
NP = 10240
RPT = NP // NS

_MESH = dict(core_axis_name="c", subcore_axis_name="s",
             num_cores=NC, num_subcores=NS)


def _deg_call(dst3d, zpad):
    @functools.partial(
        pl.kernel,
        out_type=jax.ShapeDtypeStruct((NC, 1, NP), jnp.float32),
        mesh=plsc.VectorSubcoreMesh(**_MESH),
        scratch_types=[
            pltpu.VMEM((NG, GK, K), jnp.int32),
            pltpu.VMEM((K,), jnp.float32),
            pltpu.VMEM_SHARED((NP,), jnp.float32),
        ],
    )
    def deg_kernel(dst_hbm, zero_hbm, out_hbm, dst_v, ones_v, acc):
        cid = lax.axis_index("c")
        sid = lax.axis_index("s")
        wid = cid * NS + sid
        for j in range(K // 16):
            ones_v[pl.ds(16 * j, 16)] = jnp.full((16,), 1.0, jnp.float32)
        pltpu.sync_copy(zero_hbm.at[pl.ds(sid * RPT, RPT)],
                        acc.at[pl.ds(sid * RPT, RPT)])
        pltpu.sync_copy(dst_hbm.at[wid], dst_v)
        plsc.subcore_barrier()

        def body(g, carry):
            for j in range(GK):
                pltpu.sync_copy(ones_v, acc.at[dst_v.at[g, j]], add=True)
            return carry

        lax.fori_loop(0, NG, body, 0)
        plsc.subcore_barrier()
        pltpu.sync_copy(acc.at[pl.ds(sid * RPT, RPT)],
                        out_hbm.at[cid, 0, pl.ds(sid * RPT, RPT)])

    return deg_kernel(dst3d, zpad)


def _scatter_call(z, src3d, dst3d, zeros, width):
    @functools.partial(
        pl.kernel,
        out_type=jax.ShapeDtypeStruct((NC, NP, width), jnp.float32),
        mesh=plsc.VectorSubcoreMesh(**_MESH),
        scratch_types=[
            pltpu.VMEM((NG, GK, K), jnp.int32),
            pltpu.VMEM((NG, GK, K), jnp.int32),
            pltpu.VMEM((K, width), jnp.float32),
            pltpu.VMEM_SHARED((NP, width), jnp.float32),
            pltpu.SemaphoreType.DMA,
        ],
    )
    def scat_kernel(z_hbm, src_hbm, dst_hbm, zero_hbm, out_hbm,
                    src_v, dst_v, rows, acc, gsem):
        cid = lax.axis_index("c")
        sid = lax.axis_index("s")
        wid = cid * NS + sid

        pltpu.sync_copy(zero_hbm.at[pl.ds(sid * RPT, RPT)],
                        acc.at[pl.ds(sid * RPT, RPT)])
        pltpu.sync_copy(src_hbm.at[wid], src_v)
        pltpu.sync_copy(dst_hbm.at[wid], dst_v)
        plsc.subcore_barrier()

        def body(g, carry):
            for j in range(GK):
                pltpu.async_copy(z_hbm.at[src_v.at[g, j]], rows, gsem).wait()
                pltpu.sync_copy(rows, acc.at[dst_v.at[g, j]], add=True)
            return carry

        lax.fori_loop(0, NG, body, 0)
        plsc.subcore_barrier()
        pltpu.sync_copy(acc.at[pl.ds(sid * RPT, RPT)],
                        out_hbm.at[cid, pl.ds(sid * RPT, RPT)])

    return scat_kernel(z, src3d, dst3d, zeros)


BN = 2048


def _tc1_call(degt, xp, W1):
    def body(deg_ref, x_ref, w_ref, z_ref, dinv_ref):
        deg = deg_ref[:, 0:1] + deg_ref[:, 1:2] + 1.0
        dinv = lax.rsqrt(deg)
        dinv_ref[...] = dinv
        z_ref[...] = dinv * jnp.dot(x_ref[...], w_ref[...],
                                    preferred_element_type=jnp.float32)

    return pl.pallas_call(
        body,
        grid=(NP // BN,),
        in_specs=[
            pl.BlockSpec((BN, 2), lambda i: (i, 0)),
            pl.BlockSpec((BN, D), lambda i: (i, 0)),
            pl.BlockSpec((D, H), lambda i: (0, 0)),
        ],
        out_specs=[
            pl.BlockSpec((BN, H), lambda i: (i, 0)),
            pl.BlockSpec((BN, 1), lambda i: (i, 0)),
        ],
        out_shape=[
            jax.ShapeDtypeStruct((NP, H), jnp.float32),
            jax.ShapeDtypeStruct((NP, 1), jnp.float32),
        ],
    )(degt, xp, W1)


def _tc2_call(p, z1, dinv, b1):
    def body(p_ref, z1_ref, dinv_ref, b1_ref, y_ref):
        agg = p_ref[0] + p_ref[1] + z1_ref[...]
        h = jnp.maximum(dinv_ref[...] * agg + b1_ref[...], 0.0)
        y_ref[...] = dinv_ref[...] * h

    return pl.pallas_call(
        body,
        grid=(NP // BN,),
        in_specs=[
            pl.BlockSpec((NC, BN, H), lambda i: (0, i, 0)),
            pl.BlockSpec((BN, H), lambda i: (i, 0)),
            pl.BlockSpec((BN, 1), lambda i: (i, 0)),
            pl.BlockSpec((1, H), lambda i: (0, 0)),
        ],
        out_specs=pl.BlockSpec((BN, H), lambda i: (i, 0)),
        out_shape=jax.ShapeDtypeStruct((NP, H), jnp.float32),
    )(p, z1, dinv, b1)


def _tc3_call(q, y, dinv, W2, b2):
    def body(q_ref, y_ref, dinv_ref, w_ref, b2_ref, o_ref):
        agg = q_ref[0] + q_ref[1] + y_ref[...]
        o = dinv_ref[...] * jnp.dot(agg, w_ref[...],
                                    preferred_element_type=jnp.float32)
        o = o + b2_ref[...]
        m = jnp.max(o, axis=1, keepdims=True)
        s = jnp.log(jnp.sum(jnp.exp(o - m), axis=1, keepdims=True))
        o_ref[...] = o - m - s

    return pl.pallas_call(
        body,
        grid=(NP // BN,),
        in_specs=[
            pl.BlockSpec((NC, BN, H), lambda i: (0, i, 0)),
            pl.BlockSpec((BN, H), lambda i: (i, 0)),
            pl.BlockSpec((BN, 1), lambda i: (i, 0)),
            pl.BlockSpec((H, C), lambda i: (0, 0)),
            pl.BlockSpec((1, C), lambda i: (0, 0)),
        ],
        out_specs=pl.BlockSpec((BN, C), lambda i: (i, 0)),
        out_shape=jax.ShapeDtypeStruct((NP, C), jnp.float32),
    )(q, y, dinv, W2, b2)


def kernel(x, edge_index, W1, b1, W2, b2):
    ppt = NCH * K - E // NW
    pad = jnp.broadcast_to(N + jnp.arange(ppt, dtype=jnp.int32), (NW, ppt))
    src3d = jnp.concatenate(
        [edge_index[0].reshape(NW, E // NW), pad], axis=1
    ).reshape(NW, NG, GK, K)
    dst3d = jnp.concatenate(
        [edge_index[1].reshape(NW, E // NW), pad], axis=1
    ).reshape(NW, NG, GK, K)
    xp = jnp.pad(x, ((0, NP - N), (0, 0)))
    zpad = jnp.zeros((NP,), jnp.float32)
    zD = jnp.zeros((NP, H), jnp.float32)

    degp = _deg_call(dst3d, zpad)
    degt = degp.reshape(NC, NP).T
    z1, dinv = _tc1_call(degt, xp, W1)
    p = _scatter_call(z1, src3d, dst3d, zD, H)
    y = _tc2_call(p, z1, dinv, b1.reshape(1, H))
    q = _scatter_call(y, src3d, dst3d, zD, H)
    return _tc3_call(q, y, dinv, W2, b2.reshape(1, C))[:N]


# speedup vs baseline: 3.4666x; 1.2692x over previous
import functools

import jax
import jax.numpy as jnp
from jax import lax
from jax.experimental import pallas as pl
from jax.experimental.pallas import tpu as pltpu
from jax.experimental.pallas import tpu_sc as plsc

N = 10000
E = 320000
D = 128
H = 128
C = 16

NC = 2
NS = 16
NW = NC * NS
K = 128
NCH = 80
GK = 4
NG = NCH // GK
EP = NW * NCH * K
NP = 10240
RPT = NP // NS

_MESH = dict(core_axis_name="c", subcore_axis_name="s",
             num_cores=NC, num_subcores=NS)


def _deg_call(dst3d, zpad):
    @functools.partial(
        pl.kernel,
        out_type=jax.ShapeDtypeStruct((NC, 1, NP), jnp.float32),
        mesh=plsc.VectorSubcoreMesh(**_MESH),
        scratch_types=[
            pltpu.VMEM((NG, GK, K), jnp.int32),
            pltpu.VMEM((K,), jnp.float32),
            pltpu.VMEM_SHARED((NP,), jnp.float32),
        ],
    )
    def deg_kernel(dst_hbm, zero_hbm, out_hbm, dst_v, ones_v, acc):
        cid = lax.axis_index("c")
        sid = lax.axis_index("s")
        wid = cid * NS + sid
        for j in range(K // 16):
            ones_v[pl.ds(16 * j, 16)] = jnp.full((16,), 1.0, jnp.float32)
        pltpu.sync_copy(zero_hbm.at[pl.ds(sid * RPT, RPT)],
                        acc.at[pl.ds(sid * RPT, RPT)])
        pltpu.sync_copy(dst_hbm.at[wid], dst_v)
        plsc.subcore_barrier()

        def body(g, carry):
            for j in range(GK):
                pltpu.sync_copy(ones_v, acc.at[dst_v.at[g, j]], add=True)
            return carry

        lax.fori_loop(0, NG, body, 0)
        plsc.subcore_barrier()
        pltpu.sync_copy(acc.at[pl.ds(sid * RPT, RPT)],
                        out_hbm.at[cid, 0, pl.ds(sid * RPT, RPT)])

    return deg_kernel(dst3d, zpad)


def _scatter_call(z, src3d, dst3d, zeros, width):
    @functools.partial(
        pl.kernel,
        out_type=jax.ShapeDtypeStruct((NC, NP, width), jnp.float32),
        mesh=plsc.VectorSubcoreMesh(**_MESH),
        scratch_types=[
            pltpu.VMEM((GK, K), jnp.int32),
            pltpu.VMEM((GK, K), jnp.int32),
            pltpu.VMEM((GK, K), jnp.int32),
            pltpu.VMEM((GK, K), jnp.int32),
            pltpu.VMEM((K, width), jnp.float32),
            pltpu.VMEM((K, width), jnp.float32),
            pltpu.VMEM_SHARED((NP, width), jnp.float32),
            pltpu.SemaphoreType.DMA,
            pltpu.SemaphoreType.DMA,
            pltpu.SemaphoreType.DMA,
            pltpu.SemaphoreType.DMA,
        ],
    )
    def scat_kernel(z_hbm, src_hbm, dst_hbm, zero_hbm, out_hbm,
                    sa, da, sb, db, r0, r1, acc, g0, g1, isem, jsem):
        cid = lax.axis_index("c")
        sid = lax.axis_index("s")
        wid = cid * NS + sid
        rows = (r0, r1)
        gsem = (g0, g1)

        def issue_gather(sbuf, j, b):
            return pltpu.async_copy(z_hbm.at[sbuf.at[j]], rows[b], gsem[b])

        def sync_scatter(dbuf, j, b):
            pltpu.sync_copy(rows[b], acc.at[dbuf.at[j]], add=True)

        def load_idx(g, sbuf, dbuf):
            return (pltpu.async_copy(src_hbm.at[wid, g], sbuf, isem),
                    pltpu.async_copy(dst_hbm.at[wid, g], dbuf, jsem))

        pltpu.sync_copy(zero_hbm.at[pl.ds(sid * RPT, RPT)],
                        acc.at[pl.ds(sid * RPT, RPT)])
        plsc.subcore_barrier()

        for d in load_idx(0, sa, da):
            d.wait()
        issue_gather(sa, 0, 0).wait()

        def group(g, sbuf, dbuf, snext, dnext, last):
            if not last:
                di = load_idx(g + 1, snext, dnext)
            for j in range(1, GK):
                b = j % 2
                dg = issue_gather(sbuf, j, b)
                sync_scatter(dbuf, j - 1, 1 - b)
                dg.wait()
            if not last:
                for d in di:
                    d.wait()
                dg = issue_gather(snext, 0, 0)
                sync_scatter(dbuf, GK - 1, 1)
                dg.wait()
            else:
                sync_scatter(dbuf, GK - 1, 1)

        def body(i, carry):
            group(2 * i, sa, da, sb, db, False)
            group(2 * i + 1, sb, db, sa, da, False)
            return carry

        lax.fori_loop(0, NG // 2 - 1, body, 0)
        group(NG - 2, sa, da, sb, db, False)
        group(NG - 1, sb, db, sa, da, True)
        plsc.subcore_barrier()
        pltpu.sync_copy(acc.at[pl.ds(sid * RPT, RPT)],
                        out_hbm.at[cid, pl.ds(sid * RPT, RPT)])

    return scat_kernel(z, src3d, dst3d, zeros)


BN = 2048


def _tc1_call(degt, xp, W1):
    def body(deg_ref, x_ref, w_ref, z_ref, dinv_ref):
        deg = deg_ref[:, 0:1] + deg_ref[:, 1:2] + 1.0
        dinv = lax.rsqrt(deg)
        dinv_ref[...] = dinv
        z_ref[...] = dinv * jnp.dot(x_ref[...], w_ref[...],
                                    preferred_element_type=jnp.float32)

    return pl.pallas_call(
        body,
        grid=(NP // BN,),
        in_specs=[
            pl.BlockSpec((BN, 2), lambda i: (i, 0)),
            pl.BlockSpec((BN, D), lambda i: (i, 0)),
            pl.BlockSpec((D, H), lambda i: (0, 0)),
        ],
        out_specs=[
            pl.BlockSpec((BN, H), lambda i: (i, 0)),
            pl.BlockSpec((BN, 1), lambda i: (i, 0)),
        ],
        out_shape=[
            jax.ShapeDtypeStruct((NP, H), jnp.float32),
            jax.ShapeDtypeStruct((NP, 1), jnp.float32),
        ],
    )(degt, xp, W1)


def _tc2_call(p, z1, dinv, b1):
    def body(p_ref, z1_ref, dinv_ref, b1_ref, y_ref):
        agg = p_ref[0] + p_ref[1] + z1_ref[...]
        h = jnp.maximum(dinv_ref[...] * agg + b1_ref[...], 0.0)
        y_ref[...] = dinv_ref[...] * h

    return pl.pallas_call(
        body,
        grid=(NP // BN,),
        in_specs=[
            pl.BlockSpec((NC, BN, H), lambda i: (0, i, 0)),
            pl.BlockSpec((BN, H), lambda i: (i, 0)),
            pl.BlockSpec((BN, 1), lambda i: (i, 0)),
            pl.BlockSpec((1, H), lambda i: (0, 0)),
        ],
        out_specs=pl.BlockSpec((BN, H), lambda i: (i, 0)),
        out_shape=jax.ShapeDtypeStruct((NP, H), jnp.float32),
    )(p, z1, dinv, b1)


def _tc3_call(q, y, dinv, W2, b2):
    def body(q_ref, y_ref, dinv_ref, w_ref, b2_ref, o_ref):
        agg = q_ref[0] + q_ref[1] + y_ref[...]
        o = dinv_ref[...] * jnp.dot(agg, w_ref[...],
                                    preferred_element_type=jnp.float32)
        o = o + b2_ref[...]
        m = jnp.max(o, axis=1, keepdims=True)
        s = jnp.log(jnp.sum(jnp.exp(o - m), axis=1, keepdims=True))
        o_ref[...] = o - m - s

    return pl.pallas_call(
        body,
        grid=(NP // BN,),
        in_specs=[
            pl.BlockSpec((NC, BN, H), lambda i: (0, i, 0)),
            pl.BlockSpec((BN, H), lambda i: (i, 0)),
            pl.BlockSpec((BN, 1), lambda i: (i, 0)),
            pl.BlockSpec((H, C), lambda i: (0, 0)),
            pl.BlockSpec((1, C), lambda i: (0, 0)),
        ],
        out_specs=pl.BlockSpec((BN, C), lambda i: (i, 0)),
        out_shape=jax.ShapeDtypeStruct((NP, C), jnp.float32),
    )(q, y, dinv, W2, b2)


def kernel(x, edge_index, W1, b1, W2, b2):
    ppt = NCH * K - E // NW
    pad = jnp.broadcast_to(N + jnp.arange(ppt, dtype=jnp.int32), (NW, ppt))
    src3d = jnp.concatenate(
        [edge_index[0].reshape(NW, E // NW), pad], axis=1
    ).reshape(NW, NG, GK, K)
    dst3d = jnp.concatenate(
        [edge_index[1].reshape(NW, E // NW), pad], axis=1
    ).reshape(NW, NG, GK, K)
    xp = jnp.pad(x, ((0, NP - N), (0, 0)))
    zpad = jnp.zeros((NP,), jnp.float32)
    zD = jnp.zeros((NP, H), jnp.float32)

    degp = _deg_call(dst3d, zpad)
    degt = degp.reshape(NC, NP).T
    z1, dinv = _tc1_call(degt, xp, W1)
    p = _scatter_call(z1, src3d, dst3d, zD, H)
    y = _tc2_call(p, z1, dinv, b1.reshape(1, H))
    q = _scatter_call(y, src3d, dst3d, zD, H)
    return _tc3_call(q, y, dinv, W2, b2.reshape(1, C))[:N]
